# baseline (speedup 1.0000x reference)
import functools

import jax
import jax.numpy as jnp
from jax import lax
from jax.experimental import pallas as pl
from jax.experimental.pallas import tpu as pltpu, tpu_sc as plsc

N = 4096
E = 65536
DF = 128
NN = N * N
NC = 2
NS = 16
L = 16

_mesh = lambda: plsc.VectorSubcoreMesh(core_axis_name="c", subcore_axis_name="s")


def _zero_vmem(ref, n):
    z = jnp.zeros((L,), jnp.float32)

    def body(i, _):
        ref[pl.ds(i * L, L)] = z
        return 0

    lax.fori_loop(0, n // L, body, 0)


def _zero_vmem_2d_dyn(ref, rows):
    z = jnp.zeros((L,), jnp.float32)
    ncol = ref.shape[1] // L

    def body(r, _):
        for j in range(ncol):
            ref[r, pl.ds(j * L, L)] = z
        return 0

    lax.fori_loop(0, rows, body, 0)


def _build_b(edge_index):
    R = 16
    SW = N // (R * NS)
    EC = 16384

    @functools.partial(
        pl.kernel,
        out_type=jax.ShapeDtypeStruct((NC, NN), jnp.float32),
        mesh=_mesh(),
        compiler_params=pltpu.CompilerParams(needs_layout_passes=False),
        scratch_types=[
            pltpu.VMEM((EC,), jnp.int32),
            pltpu.VMEM((EC,), jnp.int32),
            pltpu.VMEM((R * N,), jnp.float32),
        ],
    )
    def k(edge_hbm, b_hbm, srcb, dstb, blk):
        c = lax.axis_index("c")
        s = lax.axis_index("s")
        onev = jnp.full((L,), 1.0, jnp.float32)
        lanes = lax.iota(jnp.int32, L)
        ebase = c * (E // NC)

        def sweep(t, _):
            row0 = t * (R * NS) + s * R
            _zero_vmem(blk, R * N)
            plsc.store_scatter(blk, [lanes * (N + 1) + row0], onev)

            def chunk(ch, _):
                pltpu.sync_copy(
                    edge_hbm.at[0, pl.ds(ebase + ch * EC, EC)], srcb)
                pltpu.sync_copy(
                    edge_hbm.at[1, pl.ds(ebase + ch * EC, EC)], dstb)

                def q16(q, _):
                    for u in range(4):
                        o = q * 4 * L + u * L
                        sv = srcb[pl.ds(o, L)]
                        dv = dstb[pl.ds(o, L)]
                        m = (dv >= row0) & (dv < row0 + R)
                        lidx = jnp.where(m, (dv - row0) * N + sv, 0)
                        plsc.store_scatter(blk, [lidx], onev, mask=m)
                    return 0

                lax.fori_loop(0, EC // (4 * L), q16, 0)
                return 0

            lax.fori_loop(0, (E // NC) // EC, chunk, 0)
            pltpu.sync_copy(blk, b_hbm.at[c, pl.ds(row0 * N, R * N)])
            return 0

        lax.fori_loop(0, SW, sweep, 0)

    return k(edge_index)


def _denom(e_flat, edge_index):
    ew = E // NS

    seg = N // NS

    @functools.partial(
        pl.kernel,
        out_type=(
            jax.ShapeDtypeStruct((E,), jnp.float32),
            jax.ShapeDtypeStruct((NC, N), jnp.float32),
            jax.ShapeDtypeStruct((NC, N), jnp.float32),
        ),
        mesh=_mesh(),
        compiler_params=pltpu.CompilerParams(needs_layout_passes=False),
        scratch_types=[
            pltpu.VMEM((ew,), jnp.int32),
            pltpu.VMEM((ew,), jnp.int32),
            pltpu.VMEM((ew,), jnp.int32),
            pltpu.VMEM((ew,), jnp.float32),
            pltpu.VMEM((N,), jnp.float32),
            pltpu.VMEM((N,), jnp.float32),
            pltpu.VMEM((NS, seg), jnp.float32),
            pltpu.VMEM((seg,), jnp.float32),
            pltpu.VMEM_SHARED((NS, N), jnp.float32),
            pltpu.VMEM_SHARED((NS, N), jnp.float32),
            pltpu.SemaphoreType.DMA,
        ],
    )
    def k(e_hbm, edge_hbm, exm_out, den_out, od_out,
          srcb, dstb, idxb, exb, dpart, opart, mstg, macc, dshr, oshr, sem):
        c = lax.axis_index("c")
        s = lax.axis_index("s")
        eoff = s * ew
        pltpu.sync_copy(edge_hbm.at[0, pl.ds(eoff, ew)], srcb)
        pltpu.sync_copy(edge_hbm.at[1, pl.ds(eoff, ew)], dstb)

        def ci(q, _):
            sv = srcb[pl.ds(q * L, L)]
            dv = dstb[pl.ds(q * L, L)]
            idxb[pl.ds(q * L, L)] = sv * N + dv
            return 0

        lax.fori_loop(0, ew // L, ci, 0)
        nch = ew // 128
        for r in range(nch):
            pltpu.async_copy(
                e_hbm.at[idxb.at[pl.ds(r * 128, 128)]],
                exb.at[pl.ds(r * 128, 128)], sem)
        for r in range(nch):
            pltpu.make_async_copy(
                e_hbm.at[idxb.at[pl.ds(r * 128, 128)]],
                exb.at[pl.ds(r * 128, 128)], sem).wait()
        _zero_vmem(dpart, N)
        _zero_vmem(opart, N)
        onev = jnp.full((L,), 1.0, jnp.float32)

        def acc(q, _):
            sv = srcb[pl.ds(q * L, L)]
            ex = exb[pl.ds(q * L, L)]
            plsc.addupdate_scatter(dpart, [sv], ex)
            plsc.addupdate_scatter(opart, [sv], onev)
            return 0

        lax.fori_loop(0, ew // L, acc, 0)

        @pl.when(c == 0)
        def _():
            pltpu.sync_copy(exb, exm_out.at[pl.ds(eoff, ew)])

        pltpu.sync_copy(dpart, dshr.at[s])
        pltpu.sync_copy(opart, oshr.at[s])
        plsc.subcore_barrier()
        for src_shr, dst_out in ((dshr, den_out), (oshr, od_out)):
            pltpu.sync_copy(src_shr.at[:, pl.ds(s * seg, seg)], mstg)
            for j in range(seg // L):
                macc[pl.ds(j * L, L)] = mstg[0, pl.ds(j * L, L)]
            for t in range(1, NS):
                for j in range(seg // L):
                    macc[pl.ds(j * L, L)] = (
                        macc[pl.ds(j * L, L)] + mstg[t, pl.ds(j * L, L)])
            pltpu.sync_copy(macc, dst_out.at[c, pl.ds(s * seg, seg)])

    return k(e_flat, edge_index)


def _cos_agg(x, edge_index, exm, den):
    ew = E // (NC * NS)

    seg = N // NS

    @functools.partial(
        pl.kernel,
        out_type=(
            jax.ShapeDtypeStruct((NC, N, DF), jnp.float32),
            jax.ShapeDtypeStruct((NC, N), jnp.float32),
        ),
        mesh=_mesh(),
        compiler_params=pltpu.CompilerParams(needs_layout_passes=False),
        scratch_types=[
            pltpu.VMEM((ew,), jnp.int32),
            pltpu.VMEM((ew,), jnp.int32),
            pltpu.VMEM((ew // 128, 128), jnp.int32),
            pltpu.VMEM((ew,), jnp.float32),
            pltpu.VMEM((N,), jnp.float32),
            pltpu.VMEM((N,), jnp.float32),
            pltpu.VMEM((128, DF), jnp.float32),
            pltpu.VMEM((NS, seg), jnp.float32),
            pltpu.VMEM((seg,), jnp.float32),
            pltpu.VMEM((128, DF), jnp.float32),
            pltpu.VMEM_SHARED((N, DF), jnp.float32),
            pltpu.VMEM_SHARED((NS, N), jnp.float32),
            pltpu.SemaphoreType.DMA,
        ],
    )
    def k(x_hbm, edge_hbm, exm_hbm, den_hbm, acc_out, ws_out,
          srcb, dstb, sidx, wtsb, dloc, wpart, xg, mstg, macc, zbuf, accsh,
          wshr, sem):
        c = lax.axis_index("c")
        s = lax.axis_index("s")
        w = c * NS + s
        eoff = w * ew
        pltpu.sync_copy(edge_hbm.at[0, pl.ds(eoff, ew)], srcb)
        pltpu.sync_copy(edge_hbm.at[1, pl.ds(eoff, ew)], dstb)
        pltpu.sync_copy(den_hbm.at[c], dloc)
        pltpu.sync_copy(exm_hbm.at[pl.ds(eoff, ew)], wtsb)
        _zero_vmem_2d_dyn(zbuf, 128)
        for j in range((N // NS) // 128):
            pltpu.sync_copy(zbuf, accsh.at[pl.ds(s * (N // NS) + j * 128, 128)])
        _zero_vmem(wpart, N)
        plsc.subcore_barrier()

        def cw(q, _):
            sv = srcb[pl.ds(q * L, L)]
            d16 = plsc.load_gather(dloc, [sv])
            wt = wtsb[pl.ds(q * L, L)] / d16
            wtsb[pl.ds(q * L, L)] = wt
            plsc.addupdate_scatter(wpart, [sv], wt)
            return 0

        lax.fori_loop(0, ew // L, cw, 0)

        for r in range(ew // 128):
            def sj(j, _):
                sidx[r, pl.ds(j * L, L)] = srcb[pl.ds(r * 128 + j * L, L)]
                return 0

            lax.fori_loop(0, 128 // L, sj, 0)

        def chunk(kk, _):
            pltpu.async_copy(
                x_hbm.at[dstb.at[pl.ds(kk * 128, 128)]], xg, sem).wait()

            def row(r, _):
                bc = plsc.load_gather(
                    wtsb, [lax.broadcast(kk * 128 + r, (L,))])
                for j in range(DF // L):
                    xg[r, pl.ds(j * L, L)] = xg[r, pl.ds(j * L, L)] * bc
                return 0

            lax.fori_loop(0, 128, row, 0)
            pltpu.sync_copy(xg, accsh.at[sidx.at[kk]], add=True)
            return 0

        lax.fori_loop(0, ew // 128, chunk, 0)

        pltpu.sync_copy(wpart, wshr.at[s])
        plsc.subcore_barrier()
        pltpu.sync_copy(wshr.at[:, pl.ds(s * seg, seg)], mstg)
        for j in range(seg // L):
            macc[pl.ds(j * L, L)] = mstg[0, pl.ds(j * L, L)]
        for t in range(1, NS):
            for j in range(seg // L):
                macc[pl.ds(j * L, L)] = (
                    macc[pl.ds(j * L, L)] + mstg[t, pl.ds(j * L, L)])
        pltpu.sync_copy(macc, ws_out.at[c, pl.ds(s * seg, seg)])

        pltpu.sync_copy(accsh.at[pl.ds(s * (N // NS), N // NS)],
                        acc_out.at[c, pl.ds(s * (N // NS), N // NS)])

    return k(x, edge_index, exm, den)


def _mp_agg(h_ego, h_cos, edge_index):
    ew = E // (NC * NS)

    @functools.partial(
        pl.kernel,
        out_type=(
            jax.ShapeDtypeStruct((NC, N, DF), jnp.float32),
            jax.ShapeDtypeStruct((NC, N, DF), jnp.float32),
        ),
        mesh=_mesh(),
        compiler_params=pltpu.CompilerParams(needs_layout_passes=False),
        scratch_types=[
            pltpu.VMEM((ew,), jnp.int32),
            pltpu.VMEM((ew,), jnp.int32),
            pltpu.VMEM((ew // 128, 128), jnp.int32),
            pltpu.VMEM((128, DF), jnp.float32),
            pltpu.VMEM((128, DF), jnp.float32),
            pltpu.VMEM((128, DF), jnp.float32),
            pltpu.VMEM_SHARED((N, DF), jnp.float32),
            pltpu.VMEM_SHARED((N, DF), jnp.float32),
            pltpu.SemaphoreType.DMA,
            pltpu.SemaphoreType.DMA,
        ],
    )
    def k(he_hbm, hc_hbm, edge_hbm, agge_out, aggc_out,
          srcb, dstb, didx, ge, gc, zbuf, acce, accc, sem, sem2):
        c = lax.axis_index("c")
        s = lax.axis_index("s")
        w = c * NS + s
        eoff = w * ew
        pltpu.sync_copy(edge_hbm.at[0, pl.ds(eoff, ew)], srcb)
        pltpu.sync_copy(edge_hbm.at[1, pl.ds(eoff, ew)], dstb)
        _zero_vmem_2d_dyn(zbuf, 128)
        rows_per_tile = N // NS
        for j in range(rows_per_tile // 128):
            pltpu.sync_copy(zbuf, acce.at[pl.ds(s * rows_per_tile + j * 128, 128)])
            pltpu.sync_copy(zbuf, accc.at[pl.ds(s * rows_per_tile + j * 128, 128)])
        for r in range(ew // 128):
            def sj(j, _):
                didx[r, pl.ds(j * L, L)] = dstb[pl.ds(r * 128 + j * L, L)]
                return 0

            lax.fori_loop(0, 128 // L, sj, 0)
        plsc.subcore_barrier()

        def chunk(kk, _):
            pltpu.async_copy(
                he_hbm.at[srcb.at[pl.ds(kk * 128, 128)]], ge, sem)
            pltpu.async_copy(
                hc_hbm.at[srcb.at[pl.ds(kk * 128, 128)]], gc, sem2)
            pltpu.make_async_copy(
                he_hbm.at[srcb.at[pl.ds(kk * 128, 128)]], ge, sem).wait()
            pltpu.sync_copy(ge, acce.at[didx.at[kk]], add=True)
            pltpu.make_async_copy(
                hc_hbm.at[srcb.at[pl.ds(kk * 128, 128)]], gc, sem2).wait()
            pltpu.sync_copy(gc, accc.at[didx.at[kk]], add=True)
            return 0

        lax.fori_loop(0, ew // 128, chunk, 0)
        plsc.subcore_barrier()
        pltpu.sync_copy(acce.at[pl.ds(s * rows_per_tile, rows_per_tile)],
                        agge_out.at[c, pl.ds(s * rows_per_tile, rows_per_tile)])
        pltpu.sync_copy(accc.at[pl.ds(s * rows_per_tile, rows_per_tile)],
                        aggc_out.at[c, pl.ds(s * rows_per_tile, rows_per_tile)])

    return k(h_ego, h_cos, edge_index)


def _pca_mirror(X, n):
    Xc = X - X.mean(axis=0, keepdims=True)
    _, _, Vt = jnp.linalg.svd(Xc, full_matrices=False)
    return Xc @ Vt[:n].T


def _kmeans_mirror(X, kk, iters=20):
    key = jax.random.key(42)
    init_idx = jax.random.choice(key, X.shape[0], shape=(kk,), replace=False)
    centers = X[init_idx]
    labels = jnp.zeros((X.shape[0],), dtype=jnp.int32)
    for _ in range(iters):
        d = ((X[:, None, :] - centers[None, :, :]) ** 2).sum(-1)
        labels = jnp.argmin(d, axis=1)
        sums = jax.ops.segment_sum(X, labels, num_segments=kk)
        cnts = jax.ops.segment_sum(jnp.ones((X.shape[0],), X.dtype), labels,
                                   num_segments=kk)
        centers = sums / jnp.clip(cnts, 1.0)[:, None]
    return labels, centers


def kernel(x, edge_index, y, W_ego, b_ego, W_cos, b_cos, W_glob, b_glob,
           W_fc, b_fc):
    n_clusters = b_fc.shape[0]
    valid = y >= 0
    cls_counts = jnp.zeros((n_clusters,), jnp.int32).at[
        jnp.where(valid, y, 0)].add(jnp.where(valid, 1, 0))
    n_uniq = (cls_counts > 0).sum()
    x = x * (n_uniq > 0).astype(x.dtype)

    xd = lax.stop_gradient(x)
    nf = _pca_mirror(xd, 10)
    labels, centers = _kmeans_mirror(nf, n_clusters)
    dist = jnp.linalg.norm(nf - centers[labels], axis=1)
    thr = jnp.median(dist)
    keep = dist <= thr

    b01 = _build_b(edge_index)
    B = (b01[0] + b01[1]).reshape(N, N)

    Bb = B.astype(jnp.bfloat16)
    p2 = jax.lax.dot_general(Bb, Bb, (((1,), (0,)), ((), ())),
                             preferred_element_type=jnp.float32)
    mtf = jnp.minimum(p2, 1.0)
    xc = jnp.concatenate([x, jnp.ones((N, 1), jnp.float32)], axis=1)
    ego_cat = mtf @ xc
    counts = ego_cat[:, DF]
    ego_feats = ego_cat[:, :DF] / counts[:, None]
    h_ego = ego_feats @ W_ego + b_ego

    normx = x / jnp.clip(jnp.linalg.norm(x, axis=1, keepdims=True), 1e-12)
    em = jnp.exp(jax.lax.dot_general(
        normx, normx, (((1,), (1,)), ((), ())),
        preferred_element_type=jnp.float32))
    exm, den, od = _denom(em.reshape(NN), edge_index)

    acc, ws = _cos_agg(x, edge_index, exm, den)
    outdeg = od[0]
    wsum = ws[0] + ws[1]
    cos_agg = acc[0] + acc[1]
    safe_wsum = jnp.where(outdeg > 0, wsum, 1.0)
    cos_feats = jnp.where(outdeg[:, None] > 0, cos_agg / safe_wsum[:, None], x)
    h_cos = cos_feats @ W_cos + b_cos

    agge, aggc = _mp_agg(h_ego, h_cos, edge_index)
    ego_enc = jax.nn.relu(agge[0] + agge[1])
    cosine_enc = jax.nn.relu(aggc[0] + aggc[1])

    global_feats = x @ W_glob + b_glob
    dominant_feats = jnp.where(keep[:, None], x, 0.0)
    combined = jnp.concatenate(
        [ego_enc, dominant_feats, cosine_enc, global_feats], axis=-1)
    return jax.nn.log_softmax(combined @ W_fc + b_fc, axis=1)


# speedup vs baseline: 1.1454x; 1.1454x over previous
import functools

import jax
import jax.numpy as jnp
from jax import lax
from jax.experimental import pallas as pl
from jax.experimental.pallas import tpu as pltpu, tpu_sc as plsc

N = 4096
E = 65536
DF = 128
NN = N * N
NC = 2
NS = 16
L = 16

_mesh = lambda: plsc.VectorSubcoreMesh(core_axis_name="c", subcore_axis_name="s")


def _zero_vmem(ref, n):
    z = jnp.zeros((L,), jnp.float32)

    def body(i, _):
        ref[pl.ds(i * L, L)] = z
        return 0

    lax.fori_loop(0, n // L, body, 0)


def _zero_vmem_2d_dyn(ref, rows):
    z = jnp.zeros((L,), jnp.float32)
    ncol = ref.shape[1] // L

    def body(r, _):
        for j in range(ncol):
            ref[r, pl.ds(j * L, L)] = z
        return 0

    lax.fori_loop(0, rows, body, 0)


def _build_b(edge_index):
    R = 16
    SW = N // (R * NC * NS)
    EC = 16384

    @functools.partial(
        pl.kernel,
        out_type=jax.ShapeDtypeStruct((NN,), jnp.float32),
        mesh=_mesh(),
        compiler_params=pltpu.CompilerParams(needs_layout_passes=False),
        scratch_types=[
            pltpu.VMEM((EC,), jnp.int32),
            pltpu.VMEM((EC,), jnp.int32),
            pltpu.VMEM((R * N,), jnp.float32),
        ],
    )
    def k(edge_hbm, b_hbm, srcb, dstb, blk):
        c = lax.axis_index("c")
        s = lax.axis_index("s")
        w = c * NS + s
        onev = jnp.full((L,), 1.0, jnp.float32)

        lanes = lax.iota(jnp.int32, L)

        def sweep(t, _):
            row0 = t * (R * NC * NS) + w * R
            _zero_vmem(blk, R * N)
            plsc.store_scatter(blk, [lanes * (N + 1) + row0], onev)

            def chunk(ch, _):
                pltpu.sync_copy(edge_hbm.at[0, pl.ds(ch * EC, EC)], srcb)
                pltpu.sync_copy(edge_hbm.at[1, pl.ds(ch * EC, EC)], dstb)

                def q16(q, _):
                    for u in range(4):
                        o = q * 4 * L + u * L
                        sv = srcb[pl.ds(o, L)]
                        dv = dstb[pl.ds(o, L)]
                        m = (dv >= row0) & (dv < row0 + R)
                        lidx = jnp.where(m, (dv - row0) * N + sv, 0)
                        plsc.store_scatter(blk, [lidx], onev, mask=m)
                    return 0

                lax.fori_loop(0, EC // (4 * L), q16, 0)
                return 0

            lax.fori_loop(0, E // EC, chunk, 0)
            pltpu.sync_copy(blk, b_hbm.at[pl.ds(row0 * N, R * N)])
            return 0

        lax.fori_loop(0, SW, sweep, 0)

    return k(edge_index)


def _denom(e_flat, edge_index):
    ew = E // NS

    seg = N // NS

    @functools.partial(
        pl.kernel,
        out_type=(
            jax.ShapeDtypeStruct((E,), jnp.float32),
            jax.ShapeDtypeStruct((NC, N), jnp.float32),
            jax.ShapeDtypeStruct((NC, N), jnp.float32),
        ),
        mesh=_mesh(),
        compiler_params=pltpu.CompilerParams(needs_layout_passes=False),
        scratch_types=[
            pltpu.VMEM((ew,), jnp.int32),
            pltpu.VMEM((ew,), jnp.int32),
            pltpu.VMEM((ew,), jnp.int32),
            pltpu.VMEM((ew,), jnp.float32),
            pltpu.VMEM((N,), jnp.float32),
            pltpu.VMEM((N,), jnp.float32),
            pltpu.VMEM((NS, seg), jnp.float32),
            pltpu.VMEM((seg,), jnp.float32),
            pltpu.VMEM_SHARED((NS, N), jnp.float32),
            pltpu.VMEM_SHARED((NS, N), jnp.float32),
            pltpu.SemaphoreType.DMA,
        ],
    )
    def k(e_hbm, edge_hbm, exm_out, den_out, od_out,
          srcb, dstb, idxb, exb, dpart, opart, mstg, macc, dshr, oshr, sem):
        c = lax.axis_index("c")
        s = lax.axis_index("s")
        eoff = s * ew
        pltpu.sync_copy(edge_hbm.at[0, pl.ds(eoff, ew)], srcb)
        pltpu.sync_copy(edge_hbm.at[1, pl.ds(eoff, ew)], dstb)

        def ci(q, _):
            sv = srcb[pl.ds(q * L, L)]
            dv = dstb[pl.ds(q * L, L)]
            idxb[pl.ds(q * L, L)] = sv * N + dv
            return 0

        lax.fori_loop(0, ew // L, ci, 0)
        nch = ew // 128
        for r in range(nch):
            pltpu.async_copy(
                e_hbm.at[idxb.at[pl.ds(r * 128, 128)]],
                exb.at[pl.ds(r * 128, 128)], sem)
        for r in range(nch):
            pltpu.make_async_copy(
                e_hbm.at[idxb.at[pl.ds(r * 128, 128)]],
                exb.at[pl.ds(r * 128, 128)], sem).wait()
        _zero_vmem(dpart, N)
        _zero_vmem(opart, N)
        onev = jnp.full((L,), 1.0, jnp.float32)

        def acc(q, _):
            sv = srcb[pl.ds(q * L, L)]
            ex = exb[pl.ds(q * L, L)]
            plsc.addupdate_scatter(dpart, [sv], ex)
            plsc.addupdate_scatter(opart, [sv], onev)
            return 0

        lax.fori_loop(0, ew // L, acc, 0)

        @pl.when(c == 0)
        def _():
            pltpu.sync_copy(exb, exm_out.at[pl.ds(eoff, ew)])

        pltpu.sync_copy(dpart, dshr.at[s])
        pltpu.sync_copy(opart, oshr.at[s])
        plsc.subcore_barrier()
        for src_shr, dst_out in ((dshr, den_out), (oshr, od_out)):
            pltpu.sync_copy(src_shr.at[:, pl.ds(s * seg, seg)], mstg)
            for j in range(seg // L):
                macc[pl.ds(j * L, L)] = mstg[0, pl.ds(j * L, L)]
            for t in range(1, NS):
                for j in range(seg // L):
                    macc[pl.ds(j * L, L)] = (
                        macc[pl.ds(j * L, L)] + mstg[t, pl.ds(j * L, L)])
            pltpu.sync_copy(macc, dst_out.at[c, pl.ds(s * seg, seg)])

    return k(e_flat, edge_index)


def _cos_agg(x, edge_index, exm, den):
    ew = E // (NC * NS)

    seg = N // NS

    @functools.partial(
        pl.kernel,
        out_type=(
            jax.ShapeDtypeStruct((NC, N, DF), jnp.float32),
            jax.ShapeDtypeStruct((NC, N), jnp.float32),
        ),
        mesh=_mesh(),
        compiler_params=pltpu.CompilerParams(needs_layout_passes=False),
        scratch_types=[
            pltpu.VMEM((ew,), jnp.int32),
            pltpu.VMEM((ew,), jnp.int32),
            pltpu.VMEM((ew // 128, 128), jnp.int32),
            pltpu.VMEM((ew,), jnp.float32),
            pltpu.VMEM((N,), jnp.float32),
            pltpu.VMEM((N,), jnp.float32),
            pltpu.VMEM((128, DF), jnp.float32),
            pltpu.VMEM((NS, seg), jnp.float32),
            pltpu.VMEM((seg,), jnp.float32),
            pltpu.VMEM((128, DF), jnp.float32),
            pltpu.VMEM_SHARED((N, DF), jnp.float32),
            pltpu.VMEM_SHARED((NS, N), jnp.float32),
            pltpu.SemaphoreType.DMA,
        ],
    )
    def k(x_hbm, edge_hbm, exm_hbm, den_hbm, acc_out, ws_out,
          srcb, dstb, sidx, wtsb, dloc, wpart, xg, mstg, macc, zbuf, accsh,
          wshr, sem):
        c = lax.axis_index("c")
        s = lax.axis_index("s")
        w = c * NS + s
        eoff = w * ew
        pltpu.sync_copy(edge_hbm.at[0, pl.ds(eoff, ew)], srcb)
        pltpu.sync_copy(edge_hbm.at[1, pl.ds(eoff, ew)], dstb)
        pltpu.sync_copy(den_hbm.at[c], dloc)
        pltpu.sync_copy(exm_hbm.at[pl.ds(eoff, ew)], wtsb)
        _zero_vmem_2d_dyn(zbuf, 128)
        for j in range((N // NS) // 128):
            pltpu.sync_copy(zbuf, accsh.at[pl.ds(s * (N // NS) + j * 128, 128)])
        _zero_vmem(wpart, N)
        plsc.subcore_barrier()

        def cw(q, _):
            sv = srcb[pl.ds(q * L, L)]
            d16 = plsc.load_gather(dloc, [sv])
            wt = wtsb[pl.ds(q * L, L)] / d16
            wtsb[pl.ds(q * L, L)] = wt
            plsc.addupdate_scatter(wpart, [sv], wt)
            return 0

        lax.fori_loop(0, ew // L, cw, 0)

        for r in range(ew // 128):
            def sj(j, _):
                sidx[r, pl.ds(j * L, L)] = srcb[pl.ds(r * 128 + j * L, L)]
                return 0

            lax.fori_loop(0, 128 // L, sj, 0)

        def chunk(kk, _):
            pltpu.async_copy(
                x_hbm.at[dstb.at[pl.ds(kk * 128, 128)]], xg, sem).wait()

            def row(r, _):
                bc = plsc.load_gather(
                    wtsb, [lax.broadcast(kk * 128 + r, (L,))])
                for j in range(DF // L):
                    xg[r, pl.ds(j * L, L)] = xg[r, pl.ds(j * L, L)] * bc
                return 0

            lax.fori_loop(0, 128, row, 0)
            pltpu.sync_copy(xg, accsh.at[sidx.at[kk]], add=True)
            return 0

        lax.fori_loop(0, ew // 128, chunk, 0)

        pltpu.sync_copy(wpart, wshr.at[s])
        plsc.subcore_barrier()
        pltpu.sync_copy(wshr.at[:, pl.ds(s * seg, seg)], mstg)
        for j in range(seg // L):
            macc[pl.ds(j * L, L)] = mstg[0, pl.ds(j * L, L)]
        for t in range(1, NS):
            for j in range(seg // L):
                macc[pl.ds(j * L, L)] = (
                    macc[pl.ds(j * L, L)] + mstg[t, pl.ds(j * L, L)])
        pltpu.sync_copy(macc, ws_out.at[c, pl.ds(s * seg, seg)])

        pltpu.sync_copy(accsh.at[pl.ds(s * (N // NS), N // NS)],
                        acc_out.at[c, pl.ds(s * (N // NS), N // NS)])

    return k(x, edge_index, exm, den)


def _mp_agg(h_ego, h_cos, edge_index):
    ew = E // (NC * NS)

    @functools.partial(
        pl.kernel,
        out_type=(
            jax.ShapeDtypeStruct((NC, N, DF), jnp.float32),
            jax.ShapeDtypeStruct((NC, N, DF), jnp.float32),
        ),
        mesh=_mesh(),
        compiler_params=pltpu.CompilerParams(needs_layout_passes=False),
        scratch_types=[
            pltpu.VMEM((ew,), jnp.int32),
            pltpu.VMEM((ew,), jnp.int32),
            pltpu.VMEM((ew // 128, 128), jnp.int32),
            pltpu.VMEM((128, DF), jnp.float32),
            pltpu.VMEM((128, DF), jnp.float32),
            pltpu.VMEM((128, DF), jnp.float32),
            pltpu.VMEM_SHARED((N, DF), jnp.float32),
            pltpu.VMEM_SHARED((N, DF), jnp.float32),
            pltpu.SemaphoreType.DMA,
            pltpu.SemaphoreType.DMA,
        ],
    )
    def k(he_hbm, hc_hbm, edge_hbm, agge_out, aggc_out,
          srcb, dstb, didx, ge, gc, zbuf, acce, accc, sem, sem2):
        c = lax.axis_index("c")
        s = lax.axis_index("s")
        w = c * NS + s
        eoff = w * ew
        pltpu.sync_copy(edge_hbm.at[0, pl.ds(eoff, ew)], srcb)
        pltpu.sync_copy(edge_hbm.at[1, pl.ds(eoff, ew)], dstb)
        _zero_vmem_2d_dyn(zbuf, 128)
        rows_per_tile = N // NS
        for j in range(rows_per_tile // 128):
            pltpu.sync_copy(zbuf, acce.at[pl.ds(s * rows_per_tile + j * 128, 128)])
            pltpu.sync_copy(zbuf, accc.at[pl.ds(s * rows_per_tile + j * 128, 128)])
        for r in range(ew // 128):
            def sj(j, _):
                didx[r, pl.ds(j * L, L)] = dstb[pl.ds(r * 128 + j * L, L)]
                return 0

            lax.fori_loop(0, 128 // L, sj, 0)
        plsc.subcore_barrier()

        def chunk(kk, _):
            pltpu.async_copy(
                he_hbm.at[srcb.at[pl.ds(kk * 128, 128)]], ge, sem)
            pltpu.async_copy(
                hc_hbm.at[srcb.at[pl.ds(kk * 128, 128)]], gc, sem2)
            pltpu.make_async_copy(
                he_hbm.at[srcb.at[pl.ds(kk * 128, 128)]], ge, sem).wait()
            pltpu.sync_copy(ge, acce.at[didx.at[kk]], add=True)
            pltpu.make_async_copy(
                hc_hbm.at[srcb.at[pl.ds(kk * 128, 128)]], gc, sem2).wait()
            pltpu.sync_copy(gc, accc.at[didx.at[kk]], add=True)
            return 0

        lax.fori_loop(0, ew // 128, chunk, 0)
        plsc.subcore_barrier()
        pltpu.sync_copy(acce.at[pl.ds(s * rows_per_tile, rows_per_tile)],
                        agge_out.at[c, pl.ds(s * rows_per_tile, rows_per_tile)])
        pltpu.sync_copy(accc.at[pl.ds(s * rows_per_tile, rows_per_tile)],
                        aggc_out.at[c, pl.ds(s * rows_per_tile, rows_per_tile)])

    return k(h_ego, h_cos, edge_index)


def _pca_mirror(X, n):
    Xc = X - X.mean(axis=0, keepdims=True)
    _, _, Vt = jnp.linalg.svd(Xc, full_matrices=False)
    return Xc @ Vt[:n].T


def _kmeans_mirror(X, kk, iters=20):
    key = jax.random.key(42)
    init_idx = jax.random.choice(key, X.shape[0], shape=(kk,), replace=False)
    centers = X[init_idx]
    labels = jnp.zeros((X.shape[0],), dtype=jnp.int32)
    for _ in range(iters):
        d = ((X[:, None, :] - centers[None, :, :]) ** 2).sum(-1)
        labels = jnp.argmin(d, axis=1)
        sums = jax.ops.segment_sum(X, labels, num_segments=kk)
        cnts = jax.ops.segment_sum(jnp.ones((X.shape[0],), X.dtype), labels,
                                   num_segments=kk)
        centers = sums / jnp.clip(cnts, 1.0)[:, None]
    return labels, centers


def kernel(x, edge_index, y, W_ego, b_ego, W_cos, b_cos, W_glob, b_glob,
           W_fc, b_fc):
    n_clusters = b_fc.shape[0]
    valid = y >= 0
    cls_counts = jnp.zeros((n_clusters,), jnp.int32).at[
        jnp.where(valid, y, 0)].add(jnp.where(valid, 1, 0))
    n_uniq = (cls_counts > 0).sum()
    x = x * (n_uniq > 0).astype(x.dtype)

    xd = lax.stop_gradient(x)
    nf = _pca_mirror(xd, 10)
    labels, centers = _kmeans_mirror(nf, n_clusters)
    dist = jnp.linalg.norm(nf - centers[labels], axis=1)
    thr = jnp.median(dist)
    keep = dist <= thr

    b_flat = _build_b(edge_index)
    B = b_flat.reshape(N, N)

    Bb = B.astype(jnp.bfloat16)
    p2 = jax.lax.dot_general(Bb, Bb, (((1,), (0,)), ((), ())),
                             preferred_element_type=jnp.float32)
    mtf = jnp.minimum(p2, 1.0)
    xc = jnp.concatenate([x, jnp.ones((N, 1), jnp.float32)], axis=1)
    ego_cat = mtf @ xc
    counts = ego_cat[:, DF]
    ego_feats = ego_cat[:, :DF] / counts[:, None]
    h_ego = ego_feats @ W_ego + b_ego

    normx = x / jnp.clip(jnp.linalg.norm(x, axis=1, keepdims=True), 1e-12)
    em = jnp.exp(jax.lax.dot_general(
        normx, normx, (((1,), (1,)), ((), ())),
        preferred_element_type=jnp.float32))
    exm, den, od = _denom(em.reshape(NN), edge_index)

    acc, ws = _cos_agg(x, edge_index, exm, den)
    outdeg = od[0]
    wsum = ws[0] + ws[1]
    cos_agg = acc[0] + acc[1]
    safe_wsum = jnp.where(outdeg > 0, wsum, 1.0)
    cos_feats = jnp.where(outdeg[:, None] > 0, cos_agg / safe_wsum[:, None], x)
    h_cos = cos_feats @ W_cos + b_cos

    agge, aggc = _mp_agg(h_ego, h_cos, edge_index)
    ego_enc = jax.nn.relu(agge[0] + agge[1])
    cosine_enc = jax.nn.relu(aggc[0] + aggc[1])

    global_feats = x @ W_glob + b_glob
    dominant_feats = jnp.where(keep[:, None], x, 0.0)
    combined = jnp.concatenate(
        [ego_enc, dominant_feats, cosine_enc, global_feats], axis=-1)
    return jax.nn.log_softmax(combined @ W_fc + b_fc, axis=1)
